# baseline (speedup 1.0000x reference)
import functools
import math

import jax
import jax.numpy as jnp
from jax import lax
from jax.experimental import pallas as pl
from jax.experimental.pallas import tpu as pltpu
from jax.experimental.pallas import tpu_sc as plsc

N = 10000
D = 256
E = 160000
K = 5
FN = 5
NPAD = 10240
CH = 64
EPAD = 163840
NCH = EPAD // CH
CPW = NCH // 16
RPW = NPAD // 16
WCH = 64


def _bern_coef(degree):
    out = []
    for k in range(degree + 1):
        row = [0.0] * (degree + 1)
        for j in range(degree - k + 1):
            row[k + j] = float(math.comb(degree, k) * math.comb(degree - k, j)
                               * (-1) ** j)
        out.append(row)
    return out


COEF = _bern_coef(K)


def _sc_body(xs, sdp, araw, curp, rs_out,
             acc,
             sd, st0, st1, st2, st3, rsbuf, rs2buf,
             gs0, gs1, gs2, gs3, ss0, ss1, ss2, ss3, ws1, ws2):
    c = lax.axis_index("c")
    s = lax.axis_index("s")
    lanes = jax.lax.iota(jnp.int32, 16)
    base = s * CPW
    sts = (st0, st1, st2, st3)
    gss = (gs0, gs1, gs2, gs3)
    sss = (ss0, ss1, ss2, ss3)

    def _fill(ref, nrows, val):
        def _f(i, _):
            for q in range(8):
                ref[i, pl.ds(q * 16, 16)] = jnp.full((16,), val, jnp.float32)
            return 0
        lax.fori_loop(0, nrows, _f, 0, unroll=2)

    def _zacc(j, _):
        pltpu.sync_copy(st0, acc.at[pl.ds(s * RPW + j * WCH, WCH)])
        return 0

    _fill(st0, WCH, 0.0)
    lax.fori_loop(0, RPW // WCH, _zacc, 0)
    plsc.subcore_barrier()
    _fill(st1, CH, 1.0)

    def _deg(jj, _):
        for p in range(4):
            @pl.when(jj > 0)
            def _():
                pltpu.make_async_copy(st1, acc.at[sd.at[2 * p]],
                                      sss[p]).wait()
            pltpu.sync_copy(sdp.at[base + 4 * jj + p], sd.at[pl.ds(2 * p, 2)])
            pltpu.async_copy(st1, acc.at[sd.at[2 * p]], sss[p], add=True)
        return 0
    lax.fori_loop(0, CPW // 4, _deg, 0)
    for p in range(4):
        pltpu.make_async_copy(st1, acc.at[sd.at[2 * p]], sss[p]).wait()
    plsc.subcore_barrier()

    def _rsch(k2, _):
        pltpu.sync_copy(acc.at[pl.ds(s * RPW + k2 * 64, 64)], st2)

        def _rs(g, _):
            rvec = jnp.zeros((16,), jnp.float32)
            for jj in range(16):
                d = st2[g * 16 + jj, pl.ds(0, 16)]
                y = jnp.full((16,), 1.0, jnp.float32)
                for _ in range(10):
                    y = jnp.where(d * y * y > 2.0, 0.5 * y, y)
                for _ in range(6):
                    y = y * (1.5 - 0.5 * d * y * y)
                y = jnp.where(d >= 0.5, y, 0.0)
                rvec = jnp.where(lanes == jj, y, rvec)
            rsbuf[pl.ds(k2 * 64 + g * 16, 16)] = rvec
            rs2buf[pl.ds(k2 * 64 + g * 16, 16)] = rvec * rvec
            return 0
        lax.fori_loop(0, 4, _rs, 0)
        return 0
    lax.fori_loop(0, RPW // 64, _rsch, 0)

    @pl.when(c == 0)
    def _():
        pltpu.sync_copy(rsbuf, rs_out.at[pl.ds(s * RPW, RPW)])

    lax.fori_loop(0, RPW // WCH, _zacc, 0)

    def _initp(k2, _):
        r = s * RPW + k2 * 64
        pltpu.sync_copy(xs.at[c, pl.ds(r, 64)], st2)

        def _grp(g, _):
            rv16 = rsbuf[pl.ds(k2 * 64 + g * 16, 16)]
            for jj in range(16):
                rv = rv16[jj]
                i = g * 16 + jj
                for q in range(8):
                    st2[i, pl.ds(q * 16, 16)] = (
                        rv * st2[i, pl.ds(q * 16, 16)])
            return 0
        lax.fori_loop(0, 4, _grp, 0)
        pltpu.sync_copy(st2, curp.at[c, pl.ds(r, 64)])
        return 0
    lax.fori_loop(0, RPW // 64, _initp, 0)
    plsc.subcore_barrier()

    def _hop(t, _):
        for p in range(4):
            pltpu.sync_copy(sdp.at[base + p], sd.at[pl.ds(2 * p, 2)])
            pltpu.async_copy(curp.at[c].at[sd.at[2 * p]], sts[p], gss[p])

        def _pipe(jj, _):
            for p in range(4):
                pltpu.make_async_copy(curp.at[c].at[sd.at[2 * p]], sts[p],
                                      gss[p]).wait()
                pltpu.async_copy(sts[p], acc.at[sd.at[2 * p + 1]], sss[p],
                                 add=True)
            for p in range(4):
                pltpu.make_async_copy(sts[p], acc.at[sd.at[2 * p + 1]],
                                      sss[p]).wait()
                cn = jnp.minimum(base + 4 * (jj + 1) + p, base + CPW - 1)
                pltpu.sync_copy(sdp.at[cn], sd.at[pl.ds(2 * p, 2)])
                pltpu.async_copy(curp.at[c].at[sd.at[2 * p]], sts[p], gss[p])
            return 0
        lax.fori_loop(0, CPW // 4, _pipe, 0)
        for p in range(4):
            pltpu.make_async_copy(curp.at[c].at[sd.at[2 * p]], sts[p],
                                  gss[p]).wait()
        plsc.subcore_barrier()

        _fill(st0, WCH, 0.0)

        def _wb(k2, _):
            r = s * RPW + k2 * WCH

            @pl.when(k2 > 0)
            def _():
                rp = s * RPW + (k2 - 1) * WCH
                pltpu.make_async_copy(st1, araw.at[t, c, pl.ds(rp, WCH)],
                                      ws1).wait()
                pltpu.make_async_copy(st2, curp.at[c, pl.ds(rp, WCH)],
                                      ws2).wait()
            pltpu.sync_copy(acc.at[pl.ds(r, WCH)], st1)
            pltpu.sync_copy(st0, acc.at[pl.ds(r, WCH)])
            pltpu.async_copy(st1, araw.at[t, c, pl.ds(r, WCH)], ws1)
            pltpu.sync_copy(curp.at[c, pl.ds(r, WCH)], st2)

            def _grp(g, _):
                rv16 = rs2buf[pl.ds(k2 * 64 + g * 16, 16)]
                for jj in range(16):
                    rv2 = rv16[jj]
                    i = g * 16 + jj
                    for q in range(8):
                        st2[i, pl.ds(q * 16, 16)] = (
                            st2[i, pl.ds(q * 16, 16)]
                            - rv2 * st1[i, pl.ds(q * 16, 16)])
                return 0
            lax.fori_loop(0, 4, _grp, 0)
            pltpu.async_copy(st2, curp.at[c, pl.ds(r, WCH)], ws2)
            return 0
        lax.fori_loop(0, RPW // WCH, _wb, 0)
        rl = s * RPW + (RPW // WCH - 1) * WCH
        pltpu.make_async_copy(st1, araw.at[t, c, pl.ds(rl, WCH)], ws1).wait()
        pltpu.make_async_copy(st2, curp.at[c, pl.ds(rl, WCH)], ws2).wait()
        plsc.subcore_barrier()
        return 0
    lax.fori_loop(0, K, _hop, 0)


def _sc_propagate(xs, sdp):
    mesh = plsc.VectorSubcoreMesh(core_axis_name="c", subcore_axis_name="s",
                                  num_cores=2, num_subcores=16)
    f = pl.kernel(
        _sc_body,
        out_type=[
            jax.ShapeDtypeStruct((K, 2, NPAD, 128), jnp.float32),
            jax.ShapeDtypeStruct((2, NPAD, 128), jnp.float32),
            jax.ShapeDtypeStruct((NPAD,), jnp.float32),
        ],
        mesh=mesh,
        scratch_types=[
            pltpu.VMEM_SHARED((NPAD, 128), jnp.float32),
            pltpu.VMEM((8, CH), jnp.int32),
            pltpu.VMEM((CH, 128), jnp.float32),
            pltpu.VMEM((CH, 128), jnp.float32),
            pltpu.VMEM((CH, 128), jnp.float32),
            pltpu.VMEM((CH, 128), jnp.float32),
            pltpu.VMEM((RPW,), jnp.float32),
            pltpu.VMEM((RPW,), jnp.float32),
            pltpu.SemaphoreType.DMA,
            pltpu.SemaphoreType.DMA,
            pltpu.SemaphoreType.DMA,
            pltpu.SemaphoreType.DMA,
            pltpu.SemaphoreType.DMA,
            pltpu.SemaphoreType.DMA,
            pltpu.SemaphoreType.DMA,
            pltpu.SemaphoreType.DMA,
            pltpu.SemaphoreType.DMA,
            pltpu.SemaphoreType.DMA,
        ],
    )
    araw, _, rs = f(xs, sdp)
    return araw, rs


def _combine_body(x_ref, a_ref, rs_ref, out_ref):
    rs = rs_ref[...]
    for h in range(2):
        bx = [x_ref[:, h * 128:(h + 1) * 128]]
        for t in range(K):
            bx.append(bx[t] - rs * a_ref[t, h])
        for k in range(K + 1):
            ck = COEF[k]
            accv = bx[0] * ck[0] if ck[0] != 0.0 else jnp.zeros_like(bx[0])
            for i in range(1, K + 1):
                if ck[i] != 0.0:
                    accv = accv + bx[i] * ck[i]
            for f in range(FN):
                out_ref[:, f, pl.ds(k * 256 + h * 128, 128)] = accv


def _tc_combine(x, araw, rs2d):
    bn = 200
    grid = (N // bn,)
    return pl.pallas_call(
        _combine_body,
        grid=grid,
        in_specs=[
            pl.BlockSpec((bn, D), lambda n: (n, 0)),
            pl.BlockSpec((K, 2, bn, 128), lambda n: (0, 0, n, 0)),
            pl.BlockSpec((bn, 1), lambda n: (n, 0)),
        ],
        out_specs=pl.BlockSpec((bn, FN, (K + 1) * D), lambda n: (n, 0, 0)),
        out_shape=jax.ShapeDtypeStruct((N, FN, (K + 1) * D), jnp.float32),
    )(x, araw, rs2d)


def kernel(x, edge_index, conv_weight):
    del conv_weight
    x = x.astype(jnp.float32)
    ei = edge_index.astype(jnp.int32)
    pad = jnp.full((EPAD - E,), N, jnp.int32)
    srcp = jnp.concatenate([ei[:, 0], pad]).reshape(NCH, CH)
    dstp = jnp.concatenate([ei[:, 1], pad]).reshape(NCH, CH)
    sdp = jnp.stack([srcp, dstp], axis=1)
    xp = jnp.concatenate([x, jnp.zeros((NPAD - N, D), jnp.float32)])
    xs = jnp.stack([xp[:, :128], xp[:, 128:]])
    araw, rs = _sc_propagate(xs, sdp)
    return _tc_combine(x, araw, rs.reshape(NPAD, 1))


# speedup vs baseline: 1.2232x; 1.2232x over previous
import functools
import math

import jax
import jax.numpy as jnp
from jax import lax
from jax.experimental import pallas as pl
from jax.experimental.pallas import tpu as pltpu
from jax.experimental.pallas import tpu_sc as plsc

N = 10000
D = 256
E = 160000
K = 5
FN = 5
NPAD = 10240
CH = 128
EPAD = 163840
NCH = EPAD // CH
CPW = NCH // 16
RPW = NPAD // 16
WCH = 64


def _bern_coef(degree):
    out = []
    for k in range(degree + 1):
        row = [0.0] * (degree + 1)
        for j in range(degree - k + 1):
            row[k + j] = float(math.comb(degree, k) * math.comb(degree - k, j)
                               * (-1) ** j)
        out.append(row)
    return out


COEF = _bern_coef(K)


def _sc_body(xs, sdp, araw, curp, rs_out,
             acc,
             sd, stage0, stage1, rsbuf, rs2buf,
             gsem0, gsem1, ws1, ws2):
    c = lax.axis_index("c")
    s = lax.axis_index("s")
    lanes = jax.lax.iota(jnp.int32, 16)
    base = s * CPW

    def _fill(ref, nrows, val):
        def _f(i, _):
            for q in range(8):
                ref[i, pl.ds(q * 16, 16)] = jnp.full((16,), val, jnp.float32)
            return 0
        lax.fori_loop(0, nrows, _f, 0, unroll=2)

    def _zacc(j, _):
        pltpu.sync_copy(stage0, acc.at[pl.ds(s * RPW + j * CH, CH)])
        return 0

    _fill(stage0, CH, 0.0)
    lax.fori_loop(0, RPW // CH, _zacc, 0)
    plsc.subcore_barrier()
    _fill(stage1, CH, 1.0)

    def _deg(jj, _):
        for p in range(2):
            sem = gsem0 if p == 0 else gsem1

            @pl.when(jj > 0)
            def _():
                pltpu.make_async_copy(stage1, acc.at[sd.at[2 * p]],
                                      sem).wait()
            pltpu.sync_copy(sdp.at[base + 2 * jj + p], sd.at[pl.ds(2 * p, 2)])
            pltpu.async_copy(stage1, acc.at[sd.at[2 * p]], sem, add=True)
        return 0
    lax.fori_loop(0, CPW // 2, _deg, 0)
    for p in range(2):
        pltpu.make_async_copy(stage1, acc.at[sd.at[2 * p]],
                              gsem0 if p == 0 else gsem1).wait()
    plsc.subcore_barrier()

    def _rsch(k2, _):
        pltpu.sync_copy(acc.at[pl.ds(s * RPW + k2 * 64, 64)],
                        stage1.at[pl.ds(0, 64)])

        def _rs(g, _):
            rvec = jnp.zeros((16,), jnp.float32)
            for jj in range(16):
                d = stage1[g * 16 + jj, pl.ds(0, 16)]
                y = jnp.full((16,), 1.0, jnp.float32)
                for _ in range(10):
                    y = jnp.where(d * y * y > 2.0, 0.5 * y, y)
                for _ in range(6):
                    y = y * (1.5 - 0.5 * d * y * y)
                y = jnp.where(d >= 0.5, y, 0.0)
                rvec = jnp.where(lanes == jj, y, rvec)
            rsbuf[pl.ds(k2 * 64 + g * 16, 16)] = rvec
            rs2buf[pl.ds(k2 * 64 + g * 16, 16)] = rvec * rvec
            return 0
        lax.fori_loop(0, 4, _rs, 0)
        return 0
    lax.fori_loop(0, RPW // 64, _rsch, 0)

    @pl.when(c == 0)
    def _():
        pltpu.sync_copy(rsbuf, rs_out.at[pl.ds(s * RPW, RPW)])

    lax.fori_loop(0, RPW // CH, _zacc, 0)

    def _initp(k2, _):
        r = s * RPW + k2 * 64
        pltpu.sync_copy(xs.at[c, pl.ds(r, 64)], stage1.at[pl.ds(0, 64)])

        def _grp(g, _):
            rv16 = rsbuf[pl.ds(k2 * 64 + g * 16, 16)]
            for jj in range(16):
                rv = rv16[jj]
                i = g * 16 + jj
                for q in range(8):
                    stage1[i, pl.ds(q * 16, 16)] = (
                        rv * stage1[i, pl.ds(q * 16, 16)])
            return 0
        lax.fori_loop(0, 4, _grp, 0)
        pltpu.sync_copy(stage1.at[pl.ds(0, 64)], curp.at[c, pl.ds(r, 64)])
        return 0
    lax.fori_loop(0, RPW // 64, _initp, 0)
    plsc.subcore_barrier()

    def _hop(t, _):
        pltpu.sync_copy(sdp.at[base], sd.at[pl.ds(0, 2)])
        pltpu.async_copy(curp.at[c].at[sd.at[0]], stage0, gsem0)

        def _pair(jj, _):
            cb = base + 2 * jj + 1
            pltpu.sync_copy(sdp.at[cb], sd.at[pl.ds(2, 2)])
            pltpu.async_copy(curp.at[c].at[sd.at[2]], stage1, gsem1)
            pltpu.make_async_copy(curp.at[c].at[sd.at[0]], stage0,
                                  gsem0).wait()
            pltpu.sync_copy(stage0, acc.at[sd.at[1]], add=True)
            cc = jnp.minimum(cb + 1, base + CPW - 1)
            pltpu.sync_copy(sdp.at[cc], sd.at[pl.ds(0, 2)])
            pltpu.async_copy(curp.at[c].at[sd.at[0]], stage0, gsem0)
            pltpu.make_async_copy(curp.at[c].at[sd.at[2]], stage1,
                                  gsem1).wait()
            pltpu.sync_copy(stage1, acc.at[sd.at[3]], add=True)
            return 0
        lax.fori_loop(0, CPW // 2, _pair, 0)
        pltpu.make_async_copy(curp.at[c].at[sd.at[0]], stage0, gsem0).wait()
        plsc.subcore_barrier()

        _fill(stage0, WCH, 0.0)

        def _wb(k2, _):
            r = s * RPW + k2 * WCH

            @pl.when(k2 > 0)
            def _():
                rp = s * RPW + (k2 - 1) * WCH
                pltpu.make_async_copy(stage0.at[pl.ds(64, WCH)],
                                      araw.at[t, c, pl.ds(rp, WCH)],
                                      ws1).wait()

                @pl.when(t < K - 1)
                def _():
                    pltpu.make_async_copy(stage1.at[pl.ds(0, WCH)],
                                          curp.at[c, pl.ds(rp, WCH)],
                                          ws2).wait()
            pltpu.sync_copy(acc.at[pl.ds(r, WCH)], stage0.at[pl.ds(64, WCH)])
            pltpu.sync_copy(stage0.at[pl.ds(0, WCH)], acc.at[pl.ds(r, WCH)])
            pltpu.async_copy(stage0.at[pl.ds(64, WCH)],
                             araw.at[t, c, pl.ds(r, WCH)], ws1)

            @pl.when(t < K - 1)
            def _():
                pltpu.sync_copy(curp.at[c, pl.ds(r, WCH)],
                                stage1.at[pl.ds(0, WCH)])

                def _grp(g, _):
                    rv16 = rs2buf[pl.ds(k2 * 64 + g * 16, 16)]
                    for jj in range(16):
                        rv2 = rv16[jj]
                        i = g * 16 + jj
                        for q in range(8):
                            stage1[i, pl.ds(q * 16, 16)] = (
                                stage1[i, pl.ds(q * 16, 16)]
                                - rv2 * stage0[64 + i, pl.ds(q * 16, 16)])
                    return 0
                lax.fori_loop(0, 4, _grp, 0)
                pltpu.async_copy(stage1.at[pl.ds(0, WCH)],
                                 curp.at[c, pl.ds(r, WCH)], ws2)
            return 0
        lax.fori_loop(0, RPW // WCH, _wb, 0)
        rl = s * RPW + (RPW // WCH - 1) * WCH
        pltpu.make_async_copy(stage0.at[pl.ds(64, WCH)],
                              araw.at[t, c, pl.ds(rl, WCH)], ws1).wait()

        @pl.when(t < K - 1)
        def _():
            pltpu.make_async_copy(stage1.at[pl.ds(0, WCH)],
                                  curp.at[c, pl.ds(rl, WCH)], ws2).wait()
        plsc.subcore_barrier()
        return 0
    lax.fori_loop(0, K, _hop, 0)


def _sc_propagate(xs, sdp):
    mesh = plsc.VectorSubcoreMesh(core_axis_name="c", subcore_axis_name="s",
                                  num_cores=2, num_subcores=16)
    f = pl.kernel(
        _sc_body,
        out_type=[
            jax.ShapeDtypeStruct((K, 2, NPAD, 128), jnp.float32),
            jax.ShapeDtypeStruct((2, NPAD, 128), jnp.float32),
            jax.ShapeDtypeStruct((NPAD,), jnp.float32),
        ],
        mesh=mesh,
        scratch_types=[
            pltpu.VMEM_SHARED((NPAD, 128), jnp.float32),
            pltpu.VMEM((4, CH), jnp.int32),
            pltpu.VMEM((CH, 128), jnp.float32),
            pltpu.VMEM((CH, 128), jnp.float32),
            pltpu.VMEM((RPW,), jnp.float32),
            pltpu.VMEM((RPW,), jnp.float32),
            pltpu.SemaphoreType.DMA,
            pltpu.SemaphoreType.DMA,
            pltpu.SemaphoreType.DMA,
            pltpu.SemaphoreType.DMA,
        ],
    )
    araw, _, rs = f(xs, sdp)
    return araw, rs


def _combine_body(x_ref, a_ref, rs_ref, out_ref):
    rs = rs_ref[...]
    for h in range(2):
        bx = [x_ref[:, h * 128:(h + 1) * 128]]
        for t in range(K):
            bx.append(bx[t] - rs * a_ref[t, h])
        for k in range(K + 1):
            ck = COEF[k]
            accv = bx[0] * ck[0] if ck[0] != 0.0 else jnp.zeros_like(bx[0])
            for i in range(1, K + 1):
                if ck[i] != 0.0:
                    accv = accv + bx[i] * ck[i]
            for f in range(FN):
                out_ref[:, f, pl.ds(k * 256 + h * 128, 128)] = accv


def _tc_combine(x, araw, rs2d):
    bn = 200
    grid = (N // bn,)
    return pl.pallas_call(
        _combine_body,
        grid=grid,
        in_specs=[
            pl.BlockSpec((bn, D), lambda n: (n, 0)),
            pl.BlockSpec((K, 2, bn, 128), lambda n: (0, 0, n, 0)),
            pl.BlockSpec((bn, 1), lambda n: (n, 0)),
        ],
        out_specs=pl.BlockSpec((bn, FN, (K + 1) * D), lambda n: (n, 0, 0)),
        out_shape=jax.ShapeDtypeStruct((N, FN, (K + 1) * D), jnp.float32),
    )(x, araw, rs2d)


def kernel(x, edge_index, conv_weight):
    del conv_weight
    x = x.astype(jnp.float32)
    ei = edge_index.astype(jnp.int32)
    pad = jnp.full((EPAD - E,), N, jnp.int32)
    srcp = jnp.concatenate([ei[:, 0], pad]).reshape(NCH, CH)
    dstp = jnp.concatenate([ei[:, 1], pad]).reshape(NCH, CH)
    sdp = jnp.stack([srcp, dstp], axis=1)
    xp = jnp.concatenate([x, jnp.zeros((NPAD - N, D), jnp.float32)])
    xs = jnp.stack([xp[:, :128], xp[:, 128:]])
    araw, rs = _sc_propagate(xs, sdp)
    return _tc_combine(x, araw, rs.reshape(NPAD, 1))
